# baseline (speedup 1.0000x reference)
import functools

import numpy as np
import jax
import jax.numpy as jnp
from jax.experimental import pallas as pl
from jax.experimental.pallas import tpu as pltpu


def _silu(x):
    return x * jax.nn.sigmoid(x)


def _bf(x):
    return x.astype(jnp.bfloat16)


def _f32(x):
    return x.astype(jnp.float32)


def _dist_nodiag(pos_bf_i, posT_bf_j, p2_i, p2T_j):
    g = jnp.dot(pos_bf_i, posT_bf_j, preferred_element_type=jnp.float32)
    d2 = p2_i + p2T_j - 2.0 * g
    return jnp.sqrt(jnp.maximum(d2, 0.0))


def _diag_dist(pos_bf, p2):
    q0 = _f32(pos_bf[:, 0:1])
    q1 = _f32(pos_bf[:, 1:2])
    q2 = _f32(pos_bf[:, 2:3])
    g_ii = q0 * q0 + q1 * q1 + q2 * q2
    d2 = p2 + p2 - 2.0 * g_ii
    return jnp.sqrt(jnp.maximum(d2, 0.0))


_CH = 128


def _pass1_body(ti_ref, tj_ref, posT_bf_j_ref, p2T_j_ref, af_ref, pos_bf_ref,
                p2_ref, W_in_ref, b_in_ref, W_h_ref, b_h_ref, ctx_ref,
                W_t1_ref, b_t1_ref, W_t2_ref, b_t2_ref, af0_ref, cf_ref,
                s_ref, tv_ref, msg_ref, h_ref, hbf_ref, abf_ref,
                *, n_total, hdim, blk, nsteps):
    k = pl.program_id(0)
    ti = ti_ref[k]
    tj = tj_ref[k]

    @pl.when(k == 0)
    def _init():
        msg_ref[...] = jnp.zeros_like(msg_ref)
        h = _silu(jnp.dot(_bf(af_ref[...]), _bf(W_in_ref[...]),
                          preferred_element_type=jnp.float32) + b_in_ref[...])
        h_ref[...] = h
        hbf_ref[...] = _bf(h)

    posT_bf_j = posT_bf_j_ref[...]
    p2T_j = p2T_j_ref[...]
    hbf_j = hbf_ref[pl.ds(tj * blk, blk), :]

    def _chunk(c, carry):
        r0 = ti * blk + c * _CH
        dist = _dist_nodiag(pos_bf_ref[pl.ds(r0, _CH), :], posT_bf_j,
                            p2_ref[pl.ds(r0, _CH), :], p2T_j)
        A_bf = _bf((1.0 / (dist + 1.0)))
        abf_ref[pl.ds(c * _CH, _CH), :] = A_bf
        msg_ref[pl.ds(r0, _CH), :] += jnp.dot(
            A_bf, hbf_j, preferred_element_type=jnp.float32)
        return carry

    jax.lax.fori_loop(0, blk // _CH, _chunk, 0, unroll=False)

    @pl.when(ti != tj)
    def _lower():
        hbf_i = hbf_ref[pl.ds(ti * blk, blk), :]
        msg_ref[pl.ds(tj * blk, blk), :] += jax.lax.dot_general(
            abf_ref[...], hbf_i, (((0,), (0,)), ((), ())),
            preferred_element_type=jnp.float32)

    @pl.when(k == nsteps - 1)
    def _fin():
        dist_ii = _diag_dist(pos_bf_ref[...], p2_ref[...])
        A_ii = _f32(_bf((1.0 / (dist_ii + 1.0))))
        msg = (msg_ref[...] - A_ii * _f32(hbf_ref[...])) * (1.0 / n_total)
        cf = _silu(jnp.dot(_bf(h_ref[...] + msg), _bf(W_h_ref[...]),
                           preferred_element_type=jnp.float32) + b_h_ref[...])
        cf_ref[...] = _bf(cf)
        s = jnp.sum(cf * cf, axis=1, keepdims=True)
        s_ref[...] = jnp.where(af0_ref[...] == 1.0, -1e30, s)
        Wt1 = W_t1_ref[...]
        t1 = _silu(jnp.dot(cf, Wt1[:hdim, :], preferred_element_type=jnp.float32)
                   + jnp.dot(ctx_ref[...], Wt1[hdim:, :],
                             preferred_element_type=jnp.float32)
                   + b_t1_ref[...])
        tv_ref[...] = jnp.dot(t1, W_t2_ref[...],
                              preferred_element_type=jnp.float32) + b_t2_ref[...]


def _pass2_body(ti_ref, tj_ref, posT_bf_j_ref, p2T_j_ref, sT_j_ref, pos_ref,
                pos_bf_ref, p2_ref, s_ref, cf_ref, cv_ref, accM_ref, accS_ref,
                wbf_ref, *, blk, nsteps):
    k = pl.program_id(0)
    ti = ti_ref[k]
    tj = tj_ref[k]

    @pl.when(k == 0)
    def _init():
        accM_ref[...] = jnp.zeros_like(accM_ref)
        accS_ref[...] = jnp.zeros_like(accS_ref)

    posT_bf_j = posT_bf_j_ref[...]
    p2T_j = p2T_j_ref[...]
    sT_j = sT_j_ref[...]
    jsl = pl.ds(tj * blk, blk)
    cf_j = cf_ref[jsl, :]
    pos_bf_j = pos_bf_ref[jsl, :]

    def _chunk(c, colsum):
        r0 = ti * blk + c * _CH
        rsl = pl.ds(r0, _CH)
        dist = _dist_nodiag(pos_bf_ref[rsl, :], posT_bf_j,
                            p2_ref[rsl, :], p2T_j)
        inv_dist = (1.0 / (dist + 1e-6))
        G = jax.lax.dot_general(cf_ref[rsl, :], cf_j, (((1,), (1,)), ((), ())),
                                preferred_element_type=jnp.float32)
        feat_pair = jnp.sqrt(jnp.maximum(s_ref[rsl, :] + sT_j + 2.0 * G, 0.0))
        w = inv_dist * feat_pair
        w_bf = _bf(w)
        wbf_ref[pl.ds(c * _CH, _CH), :] = w_bf
        accM_ref[rsl, :] += jnp.dot(w_bf, pos_bf_j,
                                    preferred_element_type=jnp.float32)
        accS_ref[rsl, :] += jnp.sum(w, axis=1, keepdims=True)
        return colsum + jnp.sum(w, axis=0, keepdims=True)

    colsum = jax.lax.fori_loop(0, blk // _CH, _chunk,
                               jnp.zeros((1, blk), jnp.float32), unroll=False)

    @pl.when(ti != tj)
    def _lower():
        accM_ref[jsl, :] += jax.lax.dot_general(
            wbf_ref[...], pos_bf_ref[pl.ds(ti * blk, blk), :],
            (((0,), (0,)), ((), ())), preferred_element_type=jnp.float32)
        accS_ref[jsl, :] += colsum.T

    @pl.when(k == nsteps - 1)
    def _fin():
        cf = _f32(cf_ref[...])
        G_ii = jnp.sum(cf * cf, axis=1, keepdims=True)
        dist_ii = _diag_dist(pos_bf_ref[...], p2_ref[...])
        inv_ii = (1.0 / (dist_ii + 1e-6))
        s = s_ref[...]
        fp_ii = jnp.sqrt(jnp.maximum(s + s + 2.0 * G_ii, 0.0))
        w_ii = inv_ii * fp_ii
        w_ii_bf = _f32(_bf(w_ii))
        accM = accM_ref[...] - w_ii_bf * _f32(pos_bf_ref[...])
        accS = accS_ref[...] - w_ii
        cv_ref[...] = accM - pos_ref[...] * accS


def kernel(atom_positions, atom_features, context_embed, W_in, b_in, W_h, b_h,
           W_t1, b_t1, W_t2, b_t2):
    N = atom_positions.shape[0]
    T = atom_features.shape[1]
    H = W_in.shape[1]
    f32 = jnp.float32

    BLK = 1024
    nb = N // BLK
    pairs = [(i, j) for i in range(nb) for j in range(i, nb)]
    ti = jnp.asarray(np.array([p[0] for p in pairs], dtype=np.int32))
    tj = jnp.asarray(np.array([p[1] for p in pairs], dtype=np.int32))
    nsteps = len(pairs)

    pos_bf = atom_positions.astype(jnp.bfloat16)
    posT_bf = pos_bf.T
    p2 = jnp.sum(atom_positions * atom_positions, axis=1)
    p2_col = p2[:, None]
    p2_row = p2[None, :]
    af0_col = atom_features[:, 0:1]

    b_in2 = b_in[None, :]
    b_h2 = b_h[None, :]
    b_t12 = b_t1[None, :]
    b_t22 = b_t2[None, :]

    full = lambda arr: pl.BlockSpec(arr.shape, lambda k, ti, tj: (0,) * arr.ndim)
    win_j = lambda h: pl.BlockSpec((h, BLK), lambda k, ti, tj: (0, tj[k]))

    cf_bf, s, tv = pl.pallas_call(
        functools.partial(_pass1_body, n_total=N, hdim=H, blk=BLK,
                          nsteps=nsteps),
        grid_spec=pltpu.PrefetchScalarGridSpec(
            num_scalar_prefetch=2,
            grid=(nsteps,),
            in_specs=[
                win_j(3),
                win_j(1),
                full(atom_features),
                full(pos_bf),
                full(p2_col),
                full(W_in),
                full(b_in2),
                full(W_h),
                full(b_h2),
                full(context_embed),
                full(W_t1),
                full(b_t12),
                full(W_t2),
                full(b_t22),
                full(af0_col),
            ],
            out_specs=[
                pl.BlockSpec((N, H), lambda k, ti, tj: (0, 0)),
                pl.BlockSpec((N, 1), lambda k, ti, tj: (0, 0)),
                pl.BlockSpec((N, T), lambda k, ti, tj: (0, 0)),
            ],
            scratch_shapes=[
                pltpu.VMEM((N, H), f32),
                pltpu.VMEM((N, H), f32),
                pltpu.VMEM((N, H), jnp.bfloat16),
                pltpu.VMEM((BLK, BLK), jnp.bfloat16),
            ],
        ),
        out_shape=[
            jax.ShapeDtypeStruct((N, H), jnp.bfloat16),
            jax.ShapeDtypeStruct((N, 1), f32),
            jax.ShapeDtypeStruct((N, T), f32),
        ],
    )(ti, tj, posT_bf, p2_row, atom_features, pos_bf, p2_col, W_in, b_in2,
      W_h, b_h2, context_embed, W_t1, b_t12, W_t2, b_t22, af0_col)

    s_row = s.reshape(1, N)

    cv = pl.pallas_call(
        functools.partial(_pass2_body, blk=BLK, nsteps=nsteps),
        grid_spec=pltpu.PrefetchScalarGridSpec(
            num_scalar_prefetch=2,
            grid=(nsteps,),
            in_specs=[
                win_j(3),
                win_j(1),
                win_j(1),
                full(atom_positions),
                full(pos_bf),
                full(p2_col),
                full(s),
                full(cf_bf),
            ],
            out_specs=pl.BlockSpec((N, 3), lambda k, ti, tj: (0, 0)),
            scratch_shapes=[
                pltpu.VMEM((N, 3), f32),
                pltpu.VMEM((N, 1), f32),
                pltpu.VMEM((BLK, BLK), jnp.bfloat16),
            ],
        ),
        out_shape=jax.ShapeDtypeStruct((N, 3), f32),
    )(ti, tj, posT_bf, p2_row, s_row, atom_positions, pos_bf, p2_col, s,
      cf_bf)

    return (cv, tv)


# speedup vs baseline: 1.3650x; 1.3650x over previous
import functools

import numpy as np
import jax
import jax.numpy as jnp
from jax.experimental import pallas as pl
from jax.experimental.pallas import tpu as pltpu


def _silu(x):
    return x * jax.nn.sigmoid(x)


def _bf(x):
    return x.astype(jnp.bfloat16)


def _f32(x):
    return x.astype(jnp.float32)


def _dist_nodiag(pos_bf_i, posT_bf_j, p2_i, p2T_j):
    g = jnp.dot(pos_bf_i, posT_bf_j, preferred_element_type=jnp.float32)
    d2 = p2_i + p2T_j - 2.0 * g
    return jnp.sqrt(jnp.maximum(d2, 0.0))


def _diag_dist(pos_bf, p2):
    q0 = _f32(pos_bf[:, 0:1])
    q1 = _f32(pos_bf[:, 1:2])
    q2 = _f32(pos_bf[:, 2:3])
    g_ii = q0 * q0 + q1 * q1 + q2 * q2
    d2 = p2 + p2 - 2.0 * g_ii
    return jnp.sqrt(jnp.maximum(d2, 0.0))


def _pass1_body(ti_ref, tj_ref, posT_bf_j_ref, p2T_j_ref, af_ref, pos_bf_ref,
                p2_ref, W_in_ref, b_in_ref, W_h_ref, b_h_ref, ctx_ref,
                W_t1_ref, b_t1_ref, W_t2_ref, b_t2_ref, af0_ref, cf_ref,
                s_ref, tv_ref, msg_ref, h_ref, hbf_ref,
                *, n_total, hdim, blk, nsteps):
    k = pl.program_id(0)
    ti = ti_ref[k]
    tj = tj_ref[k]

    @pl.when(k == 0)
    def _init():
        msg_ref[...] = jnp.zeros_like(msg_ref)
        h = _silu(jnp.dot(_bf(af_ref[...]), _bf(W_in_ref[...]),
                          preferred_element_type=jnp.float32) + b_in_ref[...])
        h_ref[...] = h
        hbf_ref[...] = _bf(h)

    isl = pl.ds(ti * blk, blk)
    jsl = pl.ds(tj * blk, blk)
    dist = _dist_nodiag(pos_bf_ref[isl, :], posT_bf_j_ref[...],
                        p2_ref[isl, :], p2T_j_ref[...])
    A_bf = _bf(1.0 / (dist + 1.0))
    msg_ref[isl, :] += jnp.dot(A_bf, hbf_ref[jsl, :],
                               preferred_element_type=jnp.float32)

    @pl.when(ti != tj)
    def _lower():
        msg_ref[jsl, :] += jax.lax.dot_general(
            A_bf, hbf_ref[isl, :], (((0,), (0,)), ((), ())),
            preferred_element_type=jnp.float32)

    @pl.when(k == nsteps - 1)
    def _fin():
        dist_ii = _diag_dist(pos_bf_ref[...], p2_ref[...])
        A_ii = _f32(_bf((1.0 / (dist_ii + 1.0))))
        msg = (msg_ref[...] - A_ii * _f32(hbf_ref[...])) * (1.0 / n_total)
        cf = _silu(jnp.dot(_bf(h_ref[...] + msg), _bf(W_h_ref[...]),
                           preferred_element_type=jnp.float32) + b_h_ref[...])
        cf_ref[...] = _bf(cf)
        s = jnp.sum(cf * cf, axis=1, keepdims=True)
        s_ref[...] = jnp.where(af0_ref[...] == 1.0, -1e30, s)
        Wt1 = W_t1_ref[...]
        t1 = _silu(jnp.dot(cf, Wt1[:hdim, :], preferred_element_type=jnp.float32)
                   + jnp.dot(ctx_ref[...], Wt1[hdim:, :],
                             preferred_element_type=jnp.float32)
                   + b_t1_ref[...])
        tv_ref[...] = jnp.dot(t1, W_t2_ref[...],
                              preferred_element_type=jnp.float32) + b_t2_ref[...]


def _pass2_body(ti_ref, tj_ref, posT_bf_j_ref, p2T_j_ref, sT_j_ref, pos_ref,
                pos_bf_ref, p2_ref, s_ref, cf_ref, cv_ref, accM_ref, accS_ref,
                *, blk, nsteps):
    k = pl.program_id(0)
    ti = ti_ref[k]
    tj = tj_ref[k]

    @pl.when(k == 0)
    def _init():
        accM_ref[...] = jnp.zeros_like(accM_ref)
        accS_ref[...] = jnp.zeros_like(accS_ref)

    isl = pl.ds(ti * blk, blk)
    jsl = pl.ds(tj * blk, blk)
    dist = _dist_nodiag(pos_bf_ref[isl, :], posT_bf_j_ref[...],
                        p2_ref[isl, :], p2T_j_ref[...])
    inv_dist = 1.0 / (dist + 1e-6)
    G = jax.lax.dot_general(cf_ref[isl, :], cf_ref[jsl, :],
                            (((1,), (1,)), ((), ())),
                            preferred_element_type=jnp.float32)
    feat_pair = jnp.sqrt(jnp.maximum(s_ref[isl, :] + sT_j_ref[...] + 2.0 * G,
                                     0.0))
    w = inv_dist * feat_pair
    w_bf = _bf(w)
    accM_ref[isl, :] += jnp.dot(w_bf, pos_bf_ref[jsl, :],
                                preferred_element_type=jnp.float32)
    accS_ref[isl, :] += jnp.sum(w, axis=1, keepdims=True)

    @pl.when(ti != tj)
    def _lower():
        accM_ref[jsl, :] += jax.lax.dot_general(
            w_bf, pos_bf_ref[isl, :], (((0,), (0,)), ((), ())),
            preferred_element_type=jnp.float32)
        accS_ref[jsl, :] += jnp.sum(w, axis=0, keepdims=True).T

    @pl.when(k == nsteps - 1)
    def _fin():
        cf = _f32(cf_ref[...])
        G_ii = jnp.sum(cf * cf, axis=1, keepdims=True)
        dist_ii = _diag_dist(pos_bf_ref[...], p2_ref[...])
        inv_ii = (1.0 / (dist_ii + 1e-6))
        s = s_ref[...]
        fp_ii = jnp.sqrt(jnp.maximum(s + s + 2.0 * G_ii, 0.0))
        w_ii = inv_ii * fp_ii
        w_ii_bf = _f32(_bf(w_ii))
        accM = accM_ref[...] - w_ii_bf * _f32(pos_bf_ref[...])
        accS = accS_ref[...] - w_ii
        cv_ref[...] = accM - pos_ref[...] * accS


def kernel(atom_positions, atom_features, context_embed, W_in, b_in, W_h, b_h,
           W_t1, b_t1, W_t2, b_t2):
    N = atom_positions.shape[0]
    T = atom_features.shape[1]
    H = W_in.shape[1]
    f32 = jnp.float32

    BLK = 1024
    nb = N // BLK
    pairs = [(i, j) for i in range(nb) for j in range(i, nb)]
    ti = jnp.asarray(np.array([p[0] for p in pairs], dtype=np.int32))
    tj = jnp.asarray(np.array([p[1] for p in pairs], dtype=np.int32))
    nsteps = len(pairs)

    pos_bf = atom_positions.astype(jnp.bfloat16)
    posT_bf = pos_bf.T
    p2 = jnp.sum(atom_positions * atom_positions, axis=1)
    p2_col = p2[:, None]
    p2_row = p2[None, :]
    af0_col = atom_features[:, 0:1]

    b_in2 = b_in[None, :]
    b_h2 = b_h[None, :]
    b_t12 = b_t1[None, :]
    b_t22 = b_t2[None, :]

    full = lambda arr: pl.BlockSpec(arr.shape, lambda k, ti, tj: (0,) * arr.ndim)
    win_j = lambda h: pl.BlockSpec((h, BLK), lambda k, ti, tj: (0, tj[k]))

    cf_bf, s, tv = pl.pallas_call(
        functools.partial(_pass1_body, n_total=N, hdim=H, blk=BLK,
                          nsteps=nsteps),
        grid_spec=pltpu.PrefetchScalarGridSpec(
            num_scalar_prefetch=2,
            grid=(nsteps,),
            in_specs=[
                win_j(3),
                win_j(1),
                full(atom_features),
                full(pos_bf),
                full(p2_col),
                full(W_in),
                full(b_in2),
                full(W_h),
                full(b_h2),
                full(context_embed),
                full(W_t1),
                full(b_t12),
                full(W_t2),
                full(b_t22),
                full(af0_col),
            ],
            out_specs=[
                pl.BlockSpec((N, H), lambda k, ti, tj: (0, 0)),
                pl.BlockSpec((N, 1), lambda k, ti, tj: (0, 0)),
                pl.BlockSpec((N, T), lambda k, ti, tj: (0, 0)),
            ],
            scratch_shapes=[
                pltpu.VMEM((N, H), f32),
                pltpu.VMEM((N, H), f32),
                pltpu.VMEM((N, H), jnp.bfloat16),
            ],
        ),
        out_shape=[
            jax.ShapeDtypeStruct((N, H), jnp.bfloat16),
            jax.ShapeDtypeStruct((N, 1), f32),
            jax.ShapeDtypeStruct((N, T), f32),
        ],
    )(ti, tj, posT_bf, p2_row, atom_features, pos_bf, p2_col, W_in, b_in2,
      W_h, b_h2, context_embed, W_t1, b_t12, W_t2, b_t22, af0_col)

    s_row = s.reshape(1, N)

    cv = pl.pallas_call(
        functools.partial(_pass2_body, blk=BLK, nsteps=nsteps),
        grid_spec=pltpu.PrefetchScalarGridSpec(
            num_scalar_prefetch=2,
            grid=(nsteps,),
            in_specs=[
                win_j(3),
                win_j(1),
                win_j(1),
                full(atom_positions),
                full(pos_bf),
                full(p2_col),
                full(s),
                full(cf_bf),
            ],
            out_specs=pl.BlockSpec((N, 3), lambda k, ti, tj: (0, 0)),
            scratch_shapes=[
                pltpu.VMEM((N, 3), f32),
                pltpu.VMEM((N, 1), f32),
            ],
        ),
        out_shape=jax.ShapeDtypeStruct((N, 3), f32),
    )(ti, tj, posT_bf, p2_row, s_row, atom_positions, pos_bf, p2_col, s,
      cf_bf)

    return (cv, tv)
